# baseline (speedup 1.0000x reference)
import functools
import jax
import jax.numpy as jnp
from jax import lax
from jax.experimental import pallas as pl
from jax.experimental.pallas import tpu as pltpu, tpu_sc as plsc

_NC, _NS = 2, 16
_NW = _NC * _NS
_S = 8


def _make_body(B, T, D):
    TC = T // _NW
    NSUB = TC // _S

    def body(x_hbm, emb_hbm, out_hbm, emb_v, x_v, in_sem, out_sem):
        wid = lax.axis_index("s") * _NC + lax.axis_index("c")
        t0 = wid * TC

        def start_in(g, buf):
            row = t0 + g * _S
            cps = [
                pltpu.make_async_copy(
                    emb_hbm.at[pl.ds(row, _S), :], emb_v.at[buf], in_sem),
                pltpu.make_async_copy(
                    x_hbm.at[:, pl.ds(row, _S), :], x_v.at[buf], in_sem),
            ]
            for c in cps:
                c.start()
            return cps

        def start_out(g, buf):
            row = t0 + g * _S
            cps = [pltpu.make_async_copy(
                x_v.at[buf],
                out_hbm.at[:, pl.ds(row, _S), :], out_sem)]
            for c in cps:
                c.start()
            return cps

        in_flight = {0: start_in(0, 0)}
        out_flight = {}
        for g in range(NSUB):
            buf = g % 2
            if g + 1 < NSUB:
                if g - 1 in out_flight:
                    for c in out_flight.pop(g - 1):
                        c.wait()
                in_flight[g + 1] = start_in(g + 1, (g + 1) % 2)
            for c in in_flight.pop(g):
                c.wait()

            def add_row(s, _):
                def add_one(j, _):
                    e = emb_v[buf, s, pl.ds(j * 16, 16)]
                    for b in range(B):
                        x_v[buf, b, s, pl.ds(j * 16, 16)] += e
                    return 0

                return lax.fori_loop(0, D // 16, add_one, 0, unroll=8)

            lax.fori_loop(0, _S, add_row, 0)
            out_flight[g] = start_out(g, buf)
        for g in sorted(out_flight):
            for c in out_flight[g]:
                c.wait()

    return body


def kernel(x, embedding):
    B, T, D = x.shape
    body = _make_body(B, T, D)
    mesh = plsc.VectorSubcoreMesh(
        core_axis_name="c", subcore_axis_name="s",
        num_cores=_NC, num_subcores=_NS)
    return pl.kernel(
        body,
        out_type=jax.ShapeDtypeStruct((B, T, D), jnp.float32),
        mesh=mesh,
        scratch_types=[
            pltpu.VMEM((2, _S, D), jnp.float32),
            pltpu.VMEM((2, B, _S, D), jnp.float32),
            pltpu.SemaphoreType.DMA,
            pltpu.SemaphoreType.DMA,
        ],
    )(x, embedding)
